# baseline (speedup 1.0000x reference)
import functools
import math

import jax
import jax.numpy as jnp
from jax import lax
from jax.experimental import pallas as pl
from jax.experimental.pallas import tpu as pltpu
from jax.experimental.pallas import tpu_sc as plsc

D_MODEL = 64
SCALE = math.sqrt(D_MODEL)

SEQ = 200
G0 = 104
NBUF = 2


@functools.lru_cache(maxsize=None)
def _build(NB):
    info = plsc.get_sparse_core_info()
    NW = info.num_cores * info.num_subcores
    b_per_w = NB // NW
    n_groups = b_per_w // NBUF

    mesh = plsc.VectorSubcoreMesh(core_axis_name="c", subcore_axis_name="s")

    @functools.partial(
        pl.kernel,
        mesh=mesh,
        compiler_params=pltpu.CompilerParams(use_tc_tiling_on_sc=False),
        out_type=jax.ShapeDtypeStruct((NB * SEQ, 2 * D_MODEL), jnp.float32),
        scratch_types=[
            pltpu.VMEM((b_per_w * SEQ,), jnp.int32),
            pltpu.VMEM((NBUF, SEQ, D_MODEL), jnp.float32),
            pltpu.VMEM((NBUF, SEQ, 2 * D_MODEL), jnp.float32),
            pltpu.SemaphoreType.DMA,
            pltpu.SemaphoreType.DMA,
            pltpu.SemaphoreType.DMA,
            pltpu.SemaphoreType.DMA,
        ],
    )
    def emb_kernel(idx_hbm, table_hbm, out_hbm, idx_v, rows_v, sc_v, *sems):
        gsem = sems[:NBUF]
        osem = sems[NBUF:]
        cid = lax.axis_index("c")
        sid = lax.axis_index("s")
        wid = sid * info.num_cores + cid
        bat_base = wid * b_per_w

        pltpu.sync_copy(
            idx_hbm.at[pl.ds(bat_base * SEQ, b_per_w * SEQ)], idx_v)

        def fire_gather(ci, s):
            for o, g in ((0, G0), (G0, SEQ - G0)):
                pltpu.async_copy(
                    table_hbm.at[idx_v.at[pl.ds(ci * SEQ + o, g)]],
                    rows_v.at[s, pl.ds(o, g)],
                    gsem[s],
                )

        def wait_gather(s):
            pltpu.make_async_copy(
                table_hbm.at[pl.ds(0, SEQ)], rows_v.at[s], gsem[s]).wait()

        def out_slice(ci):
            r0 = pl.multiple_of((bat_base + ci) * SEQ, SEQ)
            return out_hbm.at[pl.ds(r0, SEQ)]

        def wait_out(ci, s):
            pltpu.make_async_copy(
                sc_v.at[s], out_slice(ci), osem[s]).wait()

        for b in range(NBUF - 1):
            fire_gather(b, b)

        def group_body(g, carry):
            for b in range(NBUF):
                ci = g * NBUF + b
                wait_gather(b)

                def scale_row(r, carry2):
                    for d in range(D_MODEL // 16):
                        sl = pl.ds(d * 16, 16)
                        sc_v[b, r, sl] = rows_v[b, r, sl] * SCALE
                    return carry2

                lax.fori_loop(0, SEQ, scale_row, 0, unroll=8)

                pltpu.async_copy(sc_v.at[b], out_slice(ci), osem[b])

                s2 = (b + NBUF - 1) % NBUF
                @pl.when(ci >= 1)
                def _():
                    wait_out(ci - 1, s2)

                @pl.when(ci + NBUF - 1 < b_per_w)
                def _():
                    fire_gather(ci + NBUF - 1, s2)
            return carry

        lax.fori_loop(0, n_groups, group_body, 0)
        wait_out(b_per_w - 1, (b_per_w - 1) % NBUF)

    return emb_kernel


def kernel(x, pretrained_vector):
    NB, S = x.shape
    idx = x.reshape(NB * S).astype(jnp.int32)
    out = _build(NB)(idx, pretrained_vector)
    return out.reshape(NB, S, 2 * D_MODEL)[:, :, :D_MODEL]


# speedup vs baseline: 1.1438x; 1.1438x over previous
import functools
import math

import jax
import jax.numpy as jnp
from jax import lax
from jax.experimental import pallas as pl
from jax.experimental.pallas import tpu as pltpu
from jax.experimental.pallas import tpu_sc as plsc

D_MODEL = 64
SCALE = math.sqrt(D_MODEL)

SEQ = 200
G0 = 104
NBUF = 4


@functools.lru_cache(maxsize=None)
def _build(NB):
    info = plsc.get_sparse_core_info()
    NW = info.num_cores * info.num_subcores
    b_per_w = NB // NW
    n_groups = b_per_w // NBUF

    mesh = plsc.VectorSubcoreMesh(core_axis_name="c", subcore_axis_name="s")

    @functools.partial(
        pl.kernel,
        mesh=mesh,
        compiler_params=pltpu.CompilerParams(use_tc_tiling_on_sc=False),
        out_type=jax.ShapeDtypeStruct((NB, SEQ, D_MODEL), jnp.float32),
        scratch_types=[
            pltpu.VMEM((b_per_w * SEQ,), jnp.int32),
            pltpu.VMEM((NBUF, 1, SEQ, D_MODEL), jnp.float32),
            pltpu.SemaphoreType.DMA,
            pltpu.SemaphoreType.DMA,
            pltpu.SemaphoreType.DMA,
            pltpu.SemaphoreType.DMA,
            pltpu.SemaphoreType.DMA,
            pltpu.SemaphoreType.DMA,
            pltpu.SemaphoreType.DMA,
            pltpu.SemaphoreType.DMA,
        ],
    )
    def emb_kernel(idx_hbm, table_hbm, out_hbm, idx_v, rows_v, *sems):
        gsem = sems[:NBUF]
        osem = sems[NBUF:]
        cid = lax.axis_index("c")
        sid = lax.axis_index("s")
        wid = sid * info.num_cores + cid
        bat_base = wid * b_per_w

        pltpu.sync_copy(
            idx_hbm.at[pl.ds(bat_base * SEQ, b_per_w * SEQ)], idx_v)

        def fire_gather(ci, s):
            for o, g in ((0, G0), (G0, SEQ - G0)):
                pltpu.async_copy(
                    table_hbm.at[idx_v.at[pl.ds(ci * SEQ + o, g)]],
                    rows_v.at[s, 0, pl.ds(o, g)],
                    gsem[s],
                )

        def wait_gather(s):
            pltpu.make_async_copy(
                table_hbm.at[pl.ds(0, SEQ)], rows_v.at[s], gsem[s]).wait()

        def out_slice(ci):
            return out_hbm.at[pl.ds(bat_base + ci, 1)]

        def wait_out(ci, s):
            pltpu.make_async_copy(
                rows_v.at[s], out_slice(ci), osem[s]).wait()

        for b in range(NBUF - 1):
            fire_gather(b, b)

        def group_body(g, carry):
            for b in range(NBUF):
                ci = g * NBUF + b
                wait_gather(b)

                def scale_row(r, carry2):
                    for d in range(D_MODEL // 16):
                        sl = pl.ds(d * 16, 16)
                        rows_v[b, 0, r, sl] = rows_v[b, 0, r, sl] * SCALE
                    return carry2

                lax.fori_loop(0, SEQ, scale_row, 0, unroll=8)

                pltpu.async_copy(rows_v.at[b], out_slice(ci), osem[b])

                s2 = (b + NBUF - 1) % NBUF
                @pl.when(ci >= 1)
                def _():
                    wait_out(ci - 1, s2)

                @pl.when(ci + NBUF - 1 < b_per_w)
                def _():
                    fire_gather(ci + NBUF - 1, s2)
            return carry

        lax.fori_loop(0, n_groups, group_body, 0)
        wait_out(b_per_w - 1, (b_per_w - 1) % NBUF)

    return emb_kernel


def kernel(x, pretrained_vector):
    NB, S = x.shape
    idx = x.reshape(NB * S).astype(jnp.int32)
    return _build(NB)(idx, pretrained_vector)
